# baseline (speedup 1.0000x reference)
import functools

import numpy as np
import jax
import jax.numpy as jnp
from jax import lax
from jax.experimental import pallas as pl
from jax.experimental.pallas import tpu as pltpu
from jax.experimental.pallas import tpu_sc as plsc

N = 10000
NPAD = 10240
D = 128
ROW = 80
ADW = 16
NC, NS = 2, 16
NW = NC * NS
E = 320000
CHUNK = 80
CPW = 125
EPW = CHUNK * CPW
RPT = NPAD // NS

_f32 = jnp.float32


def _expand_mat(nh, c):
  return np.kron(np.eye(nh, dtype=np.float32), np.ones((c, 1), np.float32))


def _repeat_mat(nh, c):
  return np.kron(np.eye(nh, dtype=np.float32), np.ones((1, c), np.float32))


def _prep1_body(x_ref, w_ref, asf_ref, adf_ref, k_ref, t_ref, ad_ref, m_ref):
  x = x_ref[...]
  h = jnp.dot(x, w_ref[...], preferred_element_type=_f32)
  k = k_ref[...]
  as_ = jnp.dot(h, asf_ref[...] * k, preferred_element_type=_f32)
  ad_ = jnp.dot(h, adf_ref[...] * k, preferred_element_type=_f32)
  t_ref[...] = jnp.concatenate([as_, h, jnp.zeros((NPAD, 8), _f32)], axis=1)
  ad_ref[...] = jnp.concatenate([ad_, jnp.zeros((NPAD, 8), _f32)], axis=1)
  m = (jnp.max(as_, axis=0, keepdims=True)
       + jnp.max(ad_, axis=0, keepdims=True))
  m = jnp.where(m > 0, m, 0.2 * m)
  m_ref[...] = jnp.concatenate([m, m], axis=1)


def _prep2_body(acc_ref, t1_ref, ad1_ref, m1_ref, b1_ref, w2_ref,
                as2_ref, ad2_ref, r_ref, t_ref, ad_ref, m_ref):
  t1 = t1_ref[...]
  s = t1[:, 0:8] + ad1_ref[...][:, 0:8]
  wself = jnp.exp(jnp.maximum(s, 0.2 * s) - m1_ref[...][0:1, 0:8])
  a = acc_ref[0] + acc_ref[1]
  den = a[:, 0:8] + wself
  r = r_ref[...]
  num = a[:, 8:72] + jnp.dot(wself, r,
                             preferred_element_type=_f32) * t1[:, 8:72]
  deno = jnp.dot(den, r, preferred_element_type=_f32) + 1e-16
  o = num / deno + b1_ref[...]
  g = jnp.where(o > 0, o, jnp.exp(o) - 1.0)
  h2 = jnp.dot(g, w2_ref[...], preferred_element_type=_f32)
  as2 = jnp.dot(h2, as2_ref[...], preferred_element_type=_f32)
  ad2 = jnp.dot(h2, ad2_ref[...], preferred_element_type=_f32)
  t_ref[...] = jnp.concatenate([as2, h2, jnp.zeros((NPAD, 15), _f32)], axis=1)
  ad_ref[...] = jnp.concatenate([ad2, jnp.zeros((NPAD, 15), _f32)], axis=1)
  m = (jnp.max(as2, axis=0, keepdims=True)
       + jnp.max(ad2, axis=0, keepdims=True))
  m = jnp.where(m > 0, m, 0.2 * m)
  m_ref[...] = jnp.broadcast_to(m, (1, 16))


def _final_body(acc_ref, t2_ref, ad2_ref, m2_ref, b2_ref, out_ref):
  t2 = t2_ref[...]
  s = t2[0:N, 0:1] + ad2_ref[...][0:N, 0:1]
  wself = jnp.exp(jnp.maximum(s, 0.2 * s) - m2_ref[...][0:1, 0:1])
  a = acc_ref[0] + acc_ref[1]
  den = a[0:N, 0:1] + wself + 1e-16
  out_ref[...] = (a[0:N, 1:65] + wself * t2[0:N, 1:65]) / den + b2_ref[...]


_prep1 = pl.pallas_call(
    _prep1_body,
    out_shape=[
        jax.ShapeDtypeStruct((NPAD, ROW), _f32),
        jax.ShapeDtypeStruct((NPAD, ADW), _f32),
        jax.ShapeDtypeStruct((1, 16), _f32),
    ],
)

_prep2 = pl.pallas_call(
    _prep2_body,
    out_shape=[
        jax.ShapeDtypeStruct((NPAD, ROW), _f32),
        jax.ShapeDtypeStruct((NPAD, ADW), _f32),
        jax.ShapeDtypeStruct((1, 16), _f32),
    ],
)

_final = pl.pallas_call(
    _final_body,
    out_shape=jax.ShapeDtypeStruct((N, 64), _f32),
)


def _make_sc_edge_kernel(nh):
  mesh = plsc.VectorSubcoreMesh(
      core_axis_name="c", subcore_axis_name="s",
      num_cores=NC, num_subcores=NS)

  @functools.partial(
      pl.kernel,
      out_type=jax.ShapeDtypeStruct((NC, NPAD, ROW), _f32),
      mesh=mesh,
      compiler_params=pltpu.CompilerParams(use_tc_tiling_on_sc=False),
      scratch_types=[
          pltpu.VMEM((EPW,), jnp.int32),
          pltpu.VMEM((EPW,), jnp.int32),
          pltpu.VMEM((2, CHUNK, ROW), _f32),
          pltpu.VMEM((2, CHUNK, ADW), _f32),
          pltpu.VMEM((2, CHUNK, ROW), _f32),
          pltpu.VMEM((1, 16), _f32),
          pltpu.VMEM((64, ROW), _f32),
          pltpu.VMEM_SHARED((NPAD, ROW), _f32),
          pltpu.SemaphoreType.DMA,
          pltpu.SemaphoreType.DMA,
          pltpu.SemaphoreType.DMA,
          pltpu.SemaphoreType.DMA,
          pltpu.SemaphoreType.DMA,
          pltpu.SemaphoreType.DMA,
      ],
  )
  def sc_kernel(t_hbm, adt_hbm, m_hbm, src_hbm, dst_hbm, out_hbm,
                src_v, dst_v, s_v, d_v, o_v, m_v, z_v, acc,
                sga0, sga1, sgb0, sgb1, ssc0, ssc1):
    cid = lax.axis_index("c")
    sid = lax.axis_index("s")
    wid = sid * NC + cid
    base = sid * RPT

    z16 = jnp.zeros((16,), _f32)
    for col in range(ROW // 16):
      def zrow(r, carry, _col=col):
        z_v[r, pl.ds(_col * 16, 16)] = z16
        return carry
      lax.fori_loop(0, 64, zrow, 0)

    def zcopy(j, carry):
      pltpu.sync_copy(z_v, acc.at[pl.ds(base + j * 64, 64)])
      return carry
    lax.fori_loop(0, RPT // 64, zcopy, 0)

    pltpu.sync_copy(m_hbm, m_v)
    pltpu.sync_copy(src_hbm.at[pl.ds(wid * EPW, EPW)], src_v)
    pltpu.sync_copy(dst_hbm.at[pl.ds(wid * EPW, EPW)], dst_v)
    plsc.subcore_barrier()

    m = m_v[0, pl.ds(0, 16)]
    lane = lax.iota(jnp.int32, 16)
    head_mask = lane < nh
    one16 = jnp.ones((16,), _f32)
    zero16 = jnp.zeros((16,), jnp.int32)
    if nh == 8:
      perms = [jnp.where(head_mask, lane, zero16)]
      for k in (1, 2, 3, 4):
        perms.append(lax.shift_right_logical(lane + (16 * k - 8), 3))
    else:
      perms = [zero16] * 5

    sgas = (sga0, sga1)
    sgbs = (sgb0, sgb1)
    sscs = (ssc0, ssc1)

    def sidx(j):
      return src_v.at[pl.ds(j * CHUNK, CHUNK)]

    def didx(j):
      return dst_v.at[pl.ds(j * CHUNK, CHUNK)]

    def start_gather(j, b):
      pltpu.async_copy(t_hbm.at[sidx(j)], s_v.at[b], sgas[b])
      pltpu.async_copy(adt_hbm.at[didx(j)], d_v.at[b], sgbs[b])

    for b in (0, 1):
      start_gather(b, b)

    def compute_edges(b):
      @plsc.parallel_loop(0, CHUNK, unroll=4)
      def edge(e):
        v0 = s_v[b, e, pl.ds(0, 16)]
        t = v0 + d_v[b, e, pl.ds(0, 16)]
        t = jnp.maximum(t, 0.2 * t) - m
        w = jnp.exp(t)
        o_v[b, e, pl.ds(0, 16)] = (
            jnp.take_along_axis(w, perms[0], axis=0, mode="promise_in_bounds")
            * jnp.where(head_mask, one16, v0))
        for k in (1, 2, 3):
          vk = s_v[b, e, pl.ds(16 * k, 16)]
          o_v[b, e, pl.ds(16 * k, 16)] = vk * jnp.take_along_axis(
              w, perms[k], axis=0, mode="promise_in_bounds")
        v4 = s_v[b, e, pl.ds(64, 16)]
        o_v[b, e, pl.ds(64, 16)] = (
            jnp.take_along_axis(w, perms[4], axis=0, mode="promise_in_bounds")
            * jnp.where(head_mask, v4, 0.0))

    def outer(jj, carry):
      for b in (0, 1):
        j = 2 * jj + b

        @pl.when(j < CPW)
        def _body(b=b, j=j, jj=jj):
          pltpu.make_async_copy(t_hbm.at[sidx(j)], s_v.at[b],
                                sgas[b]).wait()
          pltpu.make_async_copy(adt_hbm.at[didx(j)], d_v.at[b],
                                sgbs[b]).wait()

          @pl.when(jj > 0)
          def _wait_scatter():
            pltpu.make_async_copy(o_v.at[b], acc.at[didx(j)],
                                  sscs[b]).wait()

          compute_edges(b)
          pltpu.async_copy(o_v.at[b], acc.at[didx(j)], sscs[b], add=True)

          @pl.when(j + 2 < CPW)
          def _prefetch():
            start_gather(j + 2, b)
      return carry

    lax.fori_loop(0, (CPW + 1) // 2, outer, 0)
    for b in (0, 1):
      pltpu.make_async_copy(o_v.at[b], acc.at[didx(0)], sscs[b]).wait()
    plsc.subcore_barrier()

    def wout(j, carry):
      pltpu.sync_copy(acc.at[pl.ds(base + j * 64, 64)],
                      out_hbm.at[cid, pl.ds(base + j * 64, 64)])
      return carry
    lax.fori_loop(0, RPT // 64, wout, 0)

  return sc_kernel


_sc_layer1 = _make_sc_edge_kernel(8)
_sc_layer2 = _make_sc_edge_kernel(1)


def kernel(x, edge_index, W1, a_src1, a_dst1, b1, W2, a_src2, a_dst2, b2):
  x_pad = jnp.pad(x, ((0, NPAD - N), (0, 0)))
  src = edge_index[0]
  dst = edge_index[1]

  kmat = jnp.asarray(_expand_mat(8, 8))
  rmat = jnp.asarray(_repeat_mat(8, 8))
  t1, ad1, m1 = _prep1(x_pad, W1, a_src1.reshape(64, 1),
                       a_dst1.reshape(64, 1), kmat)
  acc1 = _sc_layer1(t1, ad1, m1, src, dst)
  t2, ad2, m2 = _prep2(acc1, t1, ad1, m1, b1.reshape(1, 64), W2,
                       a_src2.reshape(64, 1), a_dst2.reshape(64, 1), rmat)
  acc2 = _sc_layer2(t2, ad2, m2, src, dst)
  return _final(acc2, t2, ad2, m2, b2.reshape(1, 64))


# speedup vs baseline: 1.1326x; 1.1326x over previous
import functools

import numpy as np
import jax
import jax.numpy as jnp
from jax import lax
from jax.experimental import pallas as pl
from jax.experimental.pallas import tpu as pltpu
from jax.experimental.pallas import tpu_sc as plsc

N = 10000
NPAD = 10240
D = 128
ROW = 80
ADW = 16
NC, NS = 2, 16
NW = NC * NS
E = 320000
CHUNK = 80
CPW = 125
NB = 3
EPW = CHUNK * CPW
RPT = NPAD // NS

_f32 = jnp.float32


def _expand_mat(nh, c):
  return np.kron(np.eye(nh, dtype=np.float32), np.ones((c, 1), np.float32))


def _repeat_mat(nh, c):
  return np.kron(np.eye(nh, dtype=np.float32), np.ones((1, c), np.float32))


def _prep1_body(x_ref, w_ref, asf_ref, adf_ref, k_ref, t_ref, ad_ref, m_ref):
  x = x_ref[...]
  h = jnp.dot(x, w_ref[...], preferred_element_type=_f32)
  k = k_ref[...]
  as_ = jnp.dot(h, asf_ref[...] * k, preferred_element_type=_f32)
  ad_ = jnp.dot(h, adf_ref[...] * k, preferred_element_type=_f32)
  t_ref[...] = jnp.concatenate([as_, h, jnp.zeros((NPAD, 8), _f32)], axis=1)
  ad_ref[...] = jnp.concatenate([ad_, jnp.zeros((NPAD, 8), _f32)], axis=1)
  m = (jnp.max(as_, axis=0, keepdims=True)
       + jnp.max(ad_, axis=0, keepdims=True))
  m = jnp.where(m > 0, m, 0.2 * m)
  m_ref[...] = jnp.concatenate([m, m], axis=1)


def _prep2_body(acc_ref, t1_ref, ad1_ref, m1_ref, b1_ref, w2_ref,
                as2_ref, ad2_ref, r_ref, t_ref, ad_ref, m_ref):
  t1 = t1_ref[...]
  s = t1[:, 0:8] + ad1_ref[...][:, 0:8]
  wself = jnp.exp(jnp.maximum(s, 0.2 * s) - m1_ref[...][0:1, 0:8])
  a = acc_ref[0] + acc_ref[1]
  den = a[:, 0:8] + wself
  r = r_ref[...]
  num = a[:, 8:72] + jnp.dot(wself, r,
                             preferred_element_type=_f32) * t1[:, 8:72]
  deno = jnp.dot(den, r, preferred_element_type=_f32) + 1e-16
  o = num / deno + b1_ref[...]
  g = jnp.where(o > 0, o, jnp.exp(o) - 1.0)
  h2 = jnp.dot(g, w2_ref[...], preferred_element_type=_f32)
  as2 = jnp.dot(h2, as2_ref[...], preferred_element_type=_f32)
  ad2 = jnp.dot(h2, ad2_ref[...], preferred_element_type=_f32)
  t_ref[...] = jnp.concatenate([as2, h2, jnp.zeros((NPAD, 15), _f32)], axis=1)
  ad_ref[...] = jnp.concatenate([ad2, jnp.zeros((NPAD, 15), _f32)], axis=1)
  m = (jnp.max(as2, axis=0, keepdims=True)
       + jnp.max(ad2, axis=0, keepdims=True))
  m = jnp.where(m > 0, m, 0.2 * m)
  m_ref[...] = jnp.broadcast_to(m, (1, 16))


def _final_body(acc_ref, t2_ref, ad2_ref, m2_ref, b2_ref, out_ref):
  t2 = t2_ref[...]
  s = t2[0:N, 0:1] + ad2_ref[...][0:N, 0:1]
  wself = jnp.exp(jnp.maximum(s, 0.2 * s) - m2_ref[...][0:1, 0:1])
  a = acc_ref[0] + acc_ref[1]
  den = a[0:N, 0:1] + wself + 1e-16
  out_ref[...] = (a[0:N, 1:65] + wself * t2[0:N, 1:65]) / den + b2_ref[...]


_prep1 = pl.pallas_call(
    _prep1_body,
    out_shape=[
        jax.ShapeDtypeStruct((NPAD, ROW), _f32),
        jax.ShapeDtypeStruct((NPAD, ADW), _f32),
        jax.ShapeDtypeStruct((1, 16), _f32),
    ],
)

_prep2 = pl.pallas_call(
    _prep2_body,
    out_shape=[
        jax.ShapeDtypeStruct((NPAD, ROW), _f32),
        jax.ShapeDtypeStruct((NPAD, ADW), _f32),
        jax.ShapeDtypeStruct((1, 16), _f32),
    ],
)

_final = pl.pallas_call(
    _final_body,
    out_shape=jax.ShapeDtypeStruct((N, 64), _f32),
)


def _make_sc_edge_kernel(nh):
  mesh = plsc.VectorSubcoreMesh(
      core_axis_name="c", subcore_axis_name="s",
      num_cores=NC, num_subcores=NS)

  @functools.partial(
      pl.kernel,
      out_type=jax.ShapeDtypeStruct((NC, NPAD, ROW), _f32),
      mesh=mesh,
      compiler_params=pltpu.CompilerParams(use_tc_tiling_on_sc=False),
      scratch_types=[
          pltpu.VMEM((EPW,), jnp.int32),
          pltpu.VMEM((EPW,), jnp.int32),
          pltpu.VMEM((NB, CHUNK, ROW), _f32),
          pltpu.VMEM((NB, CHUNK, ADW), _f32),
          pltpu.VMEM((NB, CHUNK, ROW), _f32),
          pltpu.VMEM((1, 16), _f32),
          pltpu.VMEM((64, ROW), _f32),
          pltpu.VMEM_SHARED((NPAD, ROW), _f32),
          pltpu.SemaphoreType.DMA,
          [pltpu.SemaphoreType.DMA] * NB,
          [pltpu.SemaphoreType.DMA] * NB,
          [pltpu.SemaphoreType.DMA] * NB,
          pltpu.SemaphoreType.DMA,
      ],
  )
  def sc_kernel(t_hbm, adt_hbm, m_hbm, src_hbm, dst_hbm, out_hbm,
                src_v, dst_v, s_v, d_v, o_v, m_v, z_v, acc,
                sidxsem, sgas, sgbs, sscs, wsem):
    cid = lax.axis_index("c")
    sid = lax.axis_index("s")
    wid = sid * NC + cid
    base = sid * RPT

    ldi_s = pltpu.async_copy(src_hbm.at[pl.ds(wid * EPW, EPW)], src_v,
                             sidxsem)
    ldi_d = pltpu.async_copy(dst_hbm.at[pl.ds(wid * EPW, EPW)], dst_v,
                             sidxsem)
    pltpu.sync_copy(m_hbm, m_v)

    z16 = jnp.zeros((16,), _f32)
    for col in range(ROW // 16):
      def zrow(r, carry, _col=col):
        z_v[r, pl.ds(_col * 16, 16)] = z16
        return carry
      lax.fori_loop(0, 64, zrow, 0)

    def zcopy(j, carry):
      pltpu.sync_copy(z_v, acc.at[pl.ds(base + j * 64, 64)])
      return carry
    lax.fori_loop(0, RPT // 64, zcopy, 0)

    ldi_s.wait()
    ldi_d.wait()
    plsc.subcore_barrier()

    m = m_v[0, pl.ds(0, 16)]
    lane = lax.iota(jnp.int32, 16)
    head_mask = lane < nh
    one16 = jnp.ones((16,), _f32)
    zero16 = jnp.zeros((16,), jnp.int32)
    if nh == 8:
      perms = [jnp.where(head_mask, lane, zero16)]
      for k in (1, 2, 3, 4):
        perms.append(lax.shift_right_logical(lane + (16 * k - 8), 3))
    else:
      perms = [zero16] * 5

    def sidx(j):
      return src_v.at[pl.ds(j * CHUNK, CHUNK)]

    def didx(j):
      return dst_v.at[pl.ds(j * CHUNK, CHUNK)]

    def start_gather(j, b):
      pltpu.async_copy(t_hbm.at[sidx(j)], s_v.at[b], sgas[b])
      pltpu.async_copy(adt_hbm.at[didx(j)], d_v.at[b], sgbs[b])

    for b in range(NB):
      start_gather(b, b)

    def compute_edges(b):
      @plsc.parallel_loop(0, CHUNK, unroll=8)
      def edge(e):
        v0 = s_v[b, e, pl.ds(0, 16)]
        t = v0 + d_v[b, e, pl.ds(0, 16)]
        t = jnp.maximum(t, 0.2 * t) - m
        w = jnp.exp(t)
        o_v[b, e, pl.ds(0, 16)] = (
            jnp.take_along_axis(w, perms[0], axis=0, mode="promise_in_bounds")
            * jnp.where(head_mask, one16, v0))
        for k in (1, 2, 3):
          vk = s_v[b, e, pl.ds(16 * k, 16)]
          o_v[b, e, pl.ds(16 * k, 16)] = vk * jnp.take_along_axis(
              w, perms[k], axis=0, mode="promise_in_bounds")
        v4 = s_v[b, e, pl.ds(64, 16)]
        o_v[b, e, pl.ds(64, 16)] = (
            jnp.take_along_axis(w, perms[4], axis=0, mode="promise_in_bounds")
            * jnp.where(head_mask, v4, 0.0))

    def outer(jj, carry):
      for b in range(NB):
        j = NB * jj + b

        @pl.when(j < CPW)
        def _body(b=b, j=j, jj=jj):
          pltpu.make_async_copy(t_hbm.at[sidx(j)], s_v.at[b],
                                sgas[b]).wait()
          pltpu.make_async_copy(adt_hbm.at[didx(j)], d_v.at[b],
                                sgbs[b]).wait()

          @pl.when(jj > 0)
          def _wait_scatter():
            pltpu.make_async_copy(o_v.at[b], acc.at[didx(j)],
                                  sscs[b]).wait()

          compute_edges(b)
          pltpu.async_copy(o_v.at[b], acc.at[didx(j)], sscs[b], add=True)

          @pl.when(j + NB < CPW)
          def _prefetch():
            start_gather(j + NB, b)
      return carry

    lax.fori_loop(0, (CPW + NB - 1) // NB, outer, 0)
    for b in range(NB):
      pltpu.make_async_copy(o_v.at[b], acc.at[didx(0)], sscs[b]).wait()
    plsc.subcore_barrier()

    def wout(j, carry):
      pltpu.async_copy(acc.at[pl.ds(base + j * 64, 64)],
                       out_hbm.at[cid, pl.ds(base + j * 64, 64)], wsem)
      return carry
    lax.fori_loop(0, RPT // 64, wout, 0)

    def wdrain(j, carry):
      pltpu.make_async_copy(
          acc.at[pl.ds(base + j * 64, 64)],
          out_hbm.at[cid, pl.ds(base + j * 64, 64)], wsem).wait()
      return carry
    lax.fori_loop(0, RPT // 64, wdrain, 0)

  return sc_kernel


_sc_layer1 = _make_sc_edge_kernel(8)
_sc_layer2 = _make_sc_edge_kernel(1)


def kernel(x, edge_index, W1, a_src1, a_dst1, b1, W2, a_src2, a_dst2, b2):
  x_pad = jnp.pad(x, ((0, NPAD - N), (0, 0)))
  src = edge_index[0]
  dst = edge_index[1]

  kmat = jnp.asarray(_expand_mat(8, 8))
  rmat = jnp.asarray(_repeat_mat(8, 8))
  t1, ad1, m1 = _prep1(x_pad, W1, a_src1.reshape(64, 1),
                       a_dst1.reshape(64, 1), kmat)
  acc1 = _sc_layer1(t1, ad1, m1, src, dst)
  t2, ad2, m2 = _prep2(acc1, t1, ad1, m1, b1.reshape(1, 64), W2,
                       a_src2.reshape(64, 1), a_dst2.reshape(64, 1), rmat)
  acc2 = _sc_layer2(t2, ad2, m2, src, dst)
  return _final(acc2, t2, ad2, m2, b2.reshape(1, 64))
